# baseline (speedup 1.0000x reference)
import functools

import jax
import jax.numpy as jnp
from jax import lax
from jax.experimental import pallas as pl
from jax.experimental.pallas import tpu as pltpu
from jax.experimental.pallas import tpu_sc as plsc

_D = 64
_SCALE = _D ** -0.5
_CHUNK = 128
_NBUF = 4


@functools.lru_cache(maxsize=None)
def _build(n_idx: int, vocab: int):
    info = plsc.get_sparse_core_info()
    nw = info.num_cores * info.num_subcores
    per_w = n_idx // nw
    assert n_idx % nw == 0 and per_w % _CHUNK == 0
    n_chunks = per_w // _CHUNK

    mesh = plsc.VectorSubcoreMesh(core_axis_name="c", subcore_axis_name="s")

    scratch = (
        [pltpu.VMEM((per_w,), jnp.int32)]
        + [pltpu.VMEM((_CHUNK, _D), jnp.float32) for _ in range(2 * _NBUF)]
        + [pltpu.SemaphoreType.DMA for _ in range(2 * _NBUF + 1)]
    )

    @functools.partial(
        pl.kernel,
        out_type=jax.ShapeDtypeStruct((n_idx, _D), jnp.float32),
        mesh=mesh,
        scratch_types=scratch,
        compiler_params=pltpu.CompilerParams(use_tc_tiling_on_sc=False),
    )
    def emb_kernel(table_hbm, x_hbm, out_hbm, *sc):
        idx_v = sc[0]
        gbufs = sc[1 : 1 + _NBUF]
        obufs = sc[1 + _NBUF : 1 + 2 * _NBUF]
        gsems = sc[1 + 2 * _NBUF : 1 + 3 * _NBUF]
        osems = sc[1 + 3 * _NBUF : 1 + 4 * _NBUF]
        isem = sc[1 + 4 * _NBUF]

        wid = lax.axis_index("s") * info.num_cores + lax.axis_index("c")
        base = wid * per_w

        pltpu.async_copy(x_hbm.at[pl.ds(base, per_w)], idx_v, isem).wait()

        def start_gather(c, b):
            pltpu.async_copy(
                table_hbm.at[idx_v.at[pl.ds(c * _CHUNK, _CHUNK)]],
                gbufs[b],
                gsems[b],
            )

        def wait_gather(b):
            pltpu.make_async_copy(
                table_hbm.at[idx_v.at[pl.ds(0, _CHUNK)]], gbufs[b], gsems[b]
            ).wait()

        def start_out(c, b):
            pltpu.async_copy(
                obufs[b], out_hbm.at[pl.ds(base + c * _CHUNK, _CHUNK)], osems[b]
            )

        def wait_out(b):
            pltpu.make_async_copy(
                obufs[b], out_hbm.at[pl.ds(0, _CHUNK)], osems[b]
            ).wait()

        for b in range(_NBUF):
            start_gather(b, b)

        def round_body(t, carry):
            for b in range(_NBUF):
                c = t * _NBUF + b
                wait_gather(b)

                @pl.when(t > 0)
                def _():
                    wait_out(b)

                def scale_row(i, _):
                    for j in range(_D // 16):
                        s = pl.ds(j * 16, 16)
                        obufs[b][i, s] = gbufs[b][i, s] * _SCALE
                    return 0

                lax.fori_loop(0, _CHUNK, scale_row, 0, unroll=4)

                @pl.when(c + _NBUF < n_chunks)
                def _():
                    start_gather(c + _NBUF, b)

                start_out(c, b)
            return carry

        lax.fori_loop(0, n_chunks // _NBUF, round_body, 0)

        for b in range(_NBUF):
            wait_out(b)

    return emb_kernel


def kernel(x, table):
    n_idx = x.shape[0] * x.shape[1]
    xflat = x.reshape(n_idx).astype(jnp.int32)
    out = _build(n_idx, table.shape[0])(table, xflat)
    return out.reshape(x.shape[0], x.shape[1], _D)
